# baseline (device time: 18591 ns/iter reference)
import jax
import jax.numpy as jnp
from jax import lax
from jax.experimental import pallas as pl
from jax.experimental.pallas import tpu as pltpu

M = 512
N = 512
C = 8
RC = M // C


def kernel(x):

    def body(
        x_ref, out_ref, sx_ref, rx_ref, sy_ref, ry_ref,
        send_sem_x, recv_sem_x, send_sem_y, recv_sem_y,
    ):
        my_x = lax.axis_index("x")
        my_y = lax.axis_index("y")
        x_nbr = (1 - my_x, my_y)
        y_nbr = (my_x, 1 - my_y)

        barrier_sem = pltpu.get_barrier_semaphore()
        pl.semaphore_signal(
            barrier_sem, inc=1, device_id=x_nbr,
            device_id_type=pl.DeviceIdType.MESH,
        )
        pl.semaphore_signal(
            barrier_sem, inc=1, device_id=y_nbr,
            device_id_type=pl.DeviceIdType.MESH,
        )
        sx_ref[...] = x_ref[0, :, :].astype(jnp.bfloat16)
        pl.semaphore_wait(barrier_sem, 2)

        rdmas_x = []
        for c in range(C):
            rows = pl.ds(c * RC, RC)
            r = pltpu.make_async_remote_copy(
                src_ref=sx_ref.at[rows],
                dst_ref=rx_ref.at[rows],
                send_sem=send_sem_x.at[c],
                recv_sem=recv_sem_x.at[c],
                device_id=x_nbr,
                device_id_type=pl.DeviceIdType.MESH,
            )
            r.start()
            rdmas_x.append(r)

        rdmas_y = []
        for c in range(C):
            rows = pl.ds(c * RC, RC)
            rdmas_x[c].wait_recv()
            sy_ref[rows] = sx_ref[rows] + rx_ref[rows]
            r = pltpu.make_async_remote_copy(
                src_ref=sy_ref.at[rows],
                dst_ref=ry_ref.at[rows],
                send_sem=send_sem_y.at[c],
                recv_sem=recv_sem_y.at[c],
                device_id=y_nbr,
                device_id_type=pl.DeviceIdType.MESH,
            )
            r.start()
            rdmas_y.append(r)
            out_ref[rows, pl.ds(my_y * N, N)] = sy_ref[rows].astype(jnp.float32)
            if c >= 1:
                p = c - 1
                prows = pl.ds(p * RC, RC)
                rdmas_y[p].wait_recv()
                out_ref[prows, pl.ds((1 - my_y) * N, N)] = ry_ref[prows].astype(
                    jnp.float32
                )

        last = C - 1
        rows = pl.ds(last * RC, RC)
        rdmas_y[last].wait_recv()
        out_ref[rows, pl.ds((1 - my_y) * N, N)] = ry_ref[rows].astype(jnp.float32)

        for c in range(C):
            rdmas_x[c].wait_send()
            rdmas_y[c].wait_send()

    return pl.pallas_call(
        body,
        out_shape=jax.ShapeDtypeStruct((M, 2 * N), jnp.float32),
        in_specs=[pl.BlockSpec(memory_space=pltpu.VMEM)],
        out_specs=pl.BlockSpec(memory_space=pltpu.VMEM),
        scratch_shapes=[
            pltpu.VMEM((M, N), jnp.bfloat16),
            pltpu.VMEM((M, N), jnp.bfloat16),
            pltpu.VMEM((M, N), jnp.bfloat16),
            pltpu.VMEM((M, N), jnp.bfloat16),
            pltpu.SemaphoreType.DMA((C,)),
            pltpu.SemaphoreType.DMA((C,)),
            pltpu.SemaphoreType.DMA((C,)),
            pltpu.SemaphoreType.DMA((C,)),
        ],
        compiler_params=pltpu.CompilerParams(collective_id=0),
    )(x)


# device time: 15676 ns/iter; 1.1860x vs baseline; 1.1860x over previous
import jax
import jax.numpy as jnp
from jax import lax
from jax.experimental import pallas as pl
from jax.experimental.pallas import tpu as pltpu

M = 512
N = 512
C = 4
RC = M // C


def kernel(x):

    def body(
        x_ref, out_ref, sx_ref, rx_ref, sy_ref, ry_ref,
        send_sem_x, recv_sem_x, send_sem_y, recv_sem_y,
    ):
        my_x = lax.axis_index("x")
        my_y = lax.axis_index("y")
        x_nbr = (1 - my_x, my_y)
        y_nbr = (my_x, 1 - my_y)

        barrier_sem = pltpu.get_barrier_semaphore()
        pl.semaphore_signal(
            barrier_sem, inc=1, device_id=x_nbr,
            device_id_type=pl.DeviceIdType.MESH,
        )
        pl.semaphore_signal(
            barrier_sem, inc=1, device_id=y_nbr,
            device_id_type=pl.DeviceIdType.MESH,
        )
        sx_ref[...] = x_ref[0, :, :].astype(jnp.bfloat16)
        pl.semaphore_wait(barrier_sem, 2)

        rdmas_x = []
        for c in range(C):
            rows = pl.ds(c * RC, RC)
            r = pltpu.make_async_remote_copy(
                src_ref=sx_ref.at[rows],
                dst_ref=rx_ref.at[rows],
                send_sem=send_sem_x.at[c],
                recv_sem=recv_sem_x.at[c],
                device_id=x_nbr,
                device_id_type=pl.DeviceIdType.MESH,
            )
            r.start()
            rdmas_x.append(r)

        rdmas_y = []
        for c in range(C):
            rows = pl.ds(c * RC, RC)
            rdmas_x[c].wait_recv()
            sy_ref[rows] = sx_ref[rows] + rx_ref[rows]
            r = pltpu.make_async_remote_copy(
                src_ref=sy_ref.at[rows],
                dst_ref=ry_ref.at[rows],
                send_sem=send_sem_y.at[c],
                recv_sem=recv_sem_y.at[c],
                device_id=y_nbr,
                device_id_type=pl.DeviceIdType.MESH,
            )
            r.start()
            rdmas_y.append(r)
            out_ref[rows, pl.ds(my_y * N, N)] = sy_ref[rows].astype(jnp.float32)

        for c in range(C):
            rows = pl.ds(c * RC, RC)
            rdmas_y[c].wait_recv()
            out_ref[rows, pl.ds((1 - my_y) * N, N)] = ry_ref[rows].astype(
                jnp.float32
            )

        for c in range(C):
            rdmas_x[c].wait_send()
            rdmas_y[c].wait_send()

    return pl.pallas_call(
        body,
        out_shape=jax.ShapeDtypeStruct((M, 2 * N), jnp.float32),
        in_specs=[pl.BlockSpec(memory_space=pltpu.VMEM)],
        out_specs=pl.BlockSpec(memory_space=pltpu.VMEM),
        scratch_shapes=[
            pltpu.VMEM((M, N), jnp.bfloat16),
            pltpu.VMEM((M, N), jnp.bfloat16),
            pltpu.VMEM((M, N), jnp.bfloat16),
            pltpu.VMEM((M, N), jnp.bfloat16),
            pltpu.SemaphoreType.DMA((C,)),
            pltpu.SemaphoreType.DMA((C,)),
            pltpu.SemaphoreType.DMA((C,)),
            pltpu.SemaphoreType.DMA((C,)),
        ],
        compiler_params=pltpu.CompilerParams(collective_id=0),
    )(x)


# device time: 15442 ns/iter; 1.2039x vs baseline; 1.0152x over previous
import jax
import jax.numpy as jnp
from jax import lax
from jax.experimental import pallas as pl
from jax.experimental.pallas import tpu as pltpu

M = 512
N = 512
C = 4
RC = M // C


def kernel(x):

    def body(
        x_ref, out_ref, sx_ref, rx_ref, sy_ref, ry_ref,
        send_sem_x, recv_sem_x, send_sem_y, recv_sem_y,
    ):
        my_x = lax.axis_index("x")
        my_y = lax.axis_index("y")
        x_nbr = (1 - my_x, my_y)
        y_nbr = (my_x, 1 - my_y)

        barrier_sem = pltpu.get_barrier_semaphore()
        pl.semaphore_signal(
            barrier_sem, inc=1, device_id=x_nbr,
            device_id_type=pl.DeviceIdType.MESH,
        )
        pl.semaphore_signal(
            barrier_sem, inc=1, device_id=y_nbr,
            device_id_type=pl.DeviceIdType.MESH,
        )
        sx_ref[...] = x_ref[0, :, :].astype(jnp.bfloat16)
        pl.semaphore_wait(barrier_sem, 2)

        rdmas_x = []
        for c in range(C):
            rows = pl.ds(c * RC, RC)
            r = pltpu.make_async_remote_copy(
                src_ref=sx_ref.at[rows],
                dst_ref=rx_ref.at[rows],
                send_sem=send_sem_x.at[c],
                recv_sem=recv_sem_x.at[c],
                device_id=x_nbr,
                device_id_type=pl.DeviceIdType.MESH,
            )
            r.start()
            rdmas_x.append(r)

        rdmas_y = []
        for c in range(C):
            rows = pl.ds(c * RC, RC)
            rdmas_x[c].wait_recv()
            sy_ref[rows] = sx_ref[rows] + rx_ref[rows]
            r = pltpu.make_async_remote_copy(
                src_ref=sy_ref.at[rows],
                dst_ref=ry_ref.at[rows],
                send_sem=send_sem_y.at[c],
                recv_sem=recv_sem_y.at[c],
                device_id=y_nbr,
                device_id_type=pl.DeviceIdType.MESH,
            )
            r.start()
            rdmas_y.append(r)
            out_ref[rows, pl.ds(my_y * N, N)] = sy_ref[rows]

        for c in range(C):
            rows = pl.ds(c * RC, RC)
            rdmas_y[c].wait_recv()
            out_ref[rows, pl.ds((1 - my_y) * N, N)] = ry_ref[rows]

        for c in range(C):
            rdmas_x[c].wait_send()
            rdmas_y[c].wait_send()

    return pl.pallas_call(
        body,
        out_shape=jax.ShapeDtypeStruct((M, 2 * N), jnp.bfloat16),
        in_specs=[pl.BlockSpec(memory_space=pltpu.VMEM)],
        out_specs=pl.BlockSpec(memory_space=pltpu.VMEM),
        scratch_shapes=[
            pltpu.VMEM((M, N), jnp.bfloat16),
            pltpu.VMEM((M, N), jnp.bfloat16),
            pltpu.VMEM((M, N), jnp.bfloat16),
            pltpu.VMEM((M, N), jnp.bfloat16),
            pltpu.SemaphoreType.DMA((C,)),
            pltpu.SemaphoreType.DMA((C,)),
            pltpu.SemaphoreType.DMA((C,)),
            pltpu.SemaphoreType.DMA((C,)),
        ],
        compiler_params=pltpu.CompilerParams(collective_id=0),
    )(x)


# device time: 15425 ns/iter; 1.2053x vs baseline; 1.0011x over previous
import jax
import jax.numpy as jnp
from jax import lax
from jax.experimental import pallas as pl
from jax.experimental.pallas import tpu as pltpu

M = 512
N = 512
C = 4
RC = M // C


def kernel(x):

    def body(
        x_ref, out_ref, sx_ref, rx_ref,
        send_sem_x, recv_sem_x, send_sem_y, recv_sem_y,
    ):
        my_x = lax.axis_index("x")
        my_y = lax.axis_index("y")
        x_nbr = (1 - my_x, my_y)
        y_nbr = (my_x, 1 - my_y)
        my_cols = pl.ds(my_y * N, N)

        barrier_sem = pltpu.get_barrier_semaphore()
        pl.semaphore_signal(
            barrier_sem, inc=1, device_id=x_nbr,
            device_id_type=pl.DeviceIdType.MESH,
        )
        pl.semaphore_signal(
            barrier_sem, inc=1, device_id=y_nbr,
            device_id_type=pl.DeviceIdType.MESH,
        )
        sx_ref[...] = x_ref[0, :, :].astype(jnp.bfloat16)
        pl.semaphore_wait(barrier_sem, 2)

        rdmas_x = []
        for c in range(C):
            rows = pl.ds(c * RC, RC)
            r = pltpu.make_async_remote_copy(
                src_ref=sx_ref.at[rows],
                dst_ref=rx_ref.at[rows],
                send_sem=send_sem_x.at[c],
                recv_sem=recv_sem_x.at[c],
                device_id=x_nbr,
                device_id_type=pl.DeviceIdType.MESH,
            )
            r.start()
            rdmas_x.append(r)

        rdmas_y = []
        for c in range(C):
            rows = pl.ds(c * RC, RC)
            rdmas_x[c].wait_recv()
            out_ref[rows, my_cols] = sx_ref[rows] + rx_ref[rows]
            r = pltpu.make_async_remote_copy(
                src_ref=out_ref.at[rows, my_cols],
                dst_ref=out_ref.at[rows, my_cols],
                send_sem=send_sem_y.at[c],
                recv_sem=recv_sem_y.at[c],
                device_id=y_nbr,
                device_id_type=pl.DeviceIdType.MESH,
            )
            r.start()
            rdmas_y.append(r)

        for c in range(C):
            rdmas_y[c].wait_recv()
        for c in range(C):
            rdmas_x[c].wait_send()
            rdmas_y[c].wait_send()

    return pl.pallas_call(
        body,
        out_shape=jax.ShapeDtypeStruct((M, 2 * N), jnp.bfloat16),
        in_specs=[pl.BlockSpec(memory_space=pltpu.VMEM)],
        out_specs=pl.BlockSpec(memory_space=pltpu.VMEM),
        scratch_shapes=[
            pltpu.VMEM((M, N), jnp.bfloat16),
            pltpu.VMEM((M, N), jnp.bfloat16),
            pltpu.SemaphoreType.DMA((C,)),
            pltpu.SemaphoreType.DMA((C,)),
            pltpu.SemaphoreType.DMA((C,)),
            pltpu.SemaphoreType.DMA((C,)),
        ],
        compiler_params=pltpu.CompilerParams(collective_id=0),
    )(x)


# device time: 14659 ns/iter; 1.2682x vs baseline; 1.0523x over previous
import jax
import jax.numpy as jnp
from jax import lax
from jax.experimental import pallas as pl
from jax.experimental.pallas import tpu as pltpu

M = 512
N = 512
C = 16
RC = M // C


def kernel(x):

    def body(
        x_ref, out_ref, sx_ref, rx_ref,
        send_sem_x, recv_sem_x, send_sem_y, recv_sem_y,
    ):
        my_x = lax.axis_index("x")
        my_y = lax.axis_index("y")
        x_nbr = (1 - my_x, my_y)
        y_nbr = (my_x, 1 - my_y)
        my_cols = pl.ds(my_y * N, N)

        barrier_sem = pltpu.get_barrier_semaphore()
        pl.semaphore_signal(
            barrier_sem, inc=1, device_id=x_nbr,
            device_id_type=pl.DeviceIdType.MESH,
        )
        pl.semaphore_signal(
            barrier_sem, inc=1, device_id=y_nbr,
            device_id_type=pl.DeviceIdType.MESH,
        )
        sx_ref[...] = x_ref[0, :, :].astype(jnp.bfloat16)
        pl.semaphore_wait(barrier_sem, 2)

        rdmas_x = []
        for c in range(C):
            rows = pl.ds(c * RC, RC)
            r = pltpu.make_async_remote_copy(
                src_ref=sx_ref.at[rows],
                dst_ref=rx_ref.at[rows],
                send_sem=send_sem_x.at[c],
                recv_sem=recv_sem_x.at[c],
                device_id=x_nbr,
                device_id_type=pl.DeviceIdType.MESH,
            )
            r.start()
            rdmas_x.append(r)

        rdmas_y = []
        for c in range(C):
            rows = pl.ds(c * RC, RC)
            rdmas_x[c].wait_recv()
            out_ref[rows, my_cols] = sx_ref[rows] + rx_ref[rows]
            r = pltpu.make_async_remote_copy(
                src_ref=out_ref.at[rows, my_cols],
                dst_ref=out_ref.at[rows, my_cols],
                send_sem=send_sem_y.at[c],
                recv_sem=recv_sem_y.at[c],
                device_id=y_nbr,
                device_id_type=pl.DeviceIdType.MESH,
            )
            r.start()
            rdmas_y.append(r)

        for c in range(C):
            rdmas_y[c].wait_recv()
        for c in range(C):
            rdmas_x[c].wait_send()
            rdmas_y[c].wait_send()

    return pl.pallas_call(
        body,
        out_shape=jax.ShapeDtypeStruct((M, 2 * N), jnp.bfloat16),
        in_specs=[pl.BlockSpec(memory_space=pltpu.VMEM)],
        out_specs=pl.BlockSpec(memory_space=pltpu.VMEM),
        scratch_shapes=[
            pltpu.VMEM((M, N), jnp.bfloat16),
            pltpu.VMEM((M, N), jnp.bfloat16),
            pltpu.SemaphoreType.DMA((C,)),
            pltpu.SemaphoreType.DMA((C,)),
            pltpu.SemaphoreType.DMA((C,)),
            pltpu.SemaphoreType.DMA((C,)),
        ],
        compiler_params=pltpu.CompilerParams(collective_id=0),
    )(x)
